# baseline (speedup 1.0000x reference)
import functools
import math

import jax
import jax.numpy as jnp
from jax import lax
from jax.experimental import pallas as pl
from jax.experimental.pallas import tpu as pltpu
from jax.experimental.pallas import tpu_sc as plsc

AVG_NUM_NEIGHBORS = 64.0
FACTOR = 1.0 / math.sqrt(AVG_NUM_NEIGHBORS)

NC = 2
NS = 16
NW = NC * NS
LANES = 16
CHUNK = 2048


def _sc_scatter_kernel(E, N):
    assert E % CHUNK == 0
    n_chunks = E // CHUNK
    chunks_per_worker = -(-n_chunks // NW)
    n_pad = -(-N // 2048) * 2048

    mesh = plsc.VectorSubcoreMesh(core_axis_name="c", subcore_axis_name="s")

    @functools.partial(
        pl.kernel,
        out_type=(
            jax.ShapeDtypeStruct((E,), jnp.float32),
            jax.ShapeDtypeStruct((NC * n_pad,), jnp.float32),
        ),
        mesh=mesh,
        compiler_params=pltpu.CompilerParams(needs_layout_passes=False),
        scratch_types=dict(
            idx_v=pltpu.VMEM((CHUNK,), jnp.int32),
            j_v=pltpu.VMEM((CHUNK,), jnp.float32),
            s_v=pltpu.VMEM((CHUNK,), jnp.float32),
            eng_v=pltpu.VMEM((CHUNK,), jnp.float32),
            acc_v=pltpu.VMEM((n_pad,), jnp.float32),
            tmp_v=pltpu.VMEM((n_pad // 8 // NS,), jnp.float32),
            res_v=pltpu.VMEM((n_pad // 8 // NS,), jnp.float32),
            acc_sh=pltpu.VMEM_SHARED((NS * (n_pad // 8),), jnp.float32),
            in_sem=pltpu.SemaphoreType.DMA,
            out_sem=pltpu.SemaphoreType.DMA,
        ),
    )
    def body(center_hbm, j_hbm, s_hbm, eng_hbm, partial_hbm,
             idx_v, j_v, s_v, eng_v, acc_v, tmp_v, res_v, acc_sh,
             in_sem, out_sem):
        cid = lax.axis_index("c")
        sid = lax.axis_index("s")
        wid = sid * NC + cid

        def zero_body(i, _):
            acc_v[pl.ds(i * LANES, LANES)] = jnp.zeros((LANES,), jnp.float32)
            return 0
        lax.fori_loop(0, n_pad // LANES, zero_body, 0, unroll=8)

        def chunk_body(k, _):
            chunk = wid + k * NW

            @pl.when(chunk < n_chunks)
            def _():
                off = chunk * CHUNK
                d1 = pltpu.async_copy(center_hbm.at[pl.ds(off, CHUNK)],
                                      idx_v, in_sem)
                d2 = pltpu.async_copy(j_hbm.at[pl.ds(off, CHUNK)], j_v, in_sem)
                d3 = pltpu.async_copy(s_hbm.at[pl.ds(off, CHUNK)], s_v, in_sem)
                d1.wait()
                d2.wait()
                d3.wait()

                def acc_body(t, _):
                    sl = pl.ds(t * LANES, LANES)
                    e16 = j_v[sl] * s_v[sl]
                    eng_v[sl] = e16
                    plsc.addupdate_scatter(acc_v, [idx_v[sl]], e16)
                    return 0
                lax.fori_loop(0, CHUNK // LANES, acc_body, 0, unroll=4)

                pltpu.async_copy(eng_v, eng_hbm.at[pl.ds(off, CHUNK)],
                                 out_sem)
                pltpu.make_async_copy(eng_v, eng_hbm.at[pl.ds(off, CHUNK)],
                                      out_sem).wait()
            return 0
        lax.fori_loop(0, chunks_per_worker, chunk_body, 0)

        half = n_pad // 8
        nps2 = half // NS

        def wave_body(w, _):
            plsc.subcore_barrier()
            pltpu.sync_copy(acc_v.at[pl.ds(w * half, half)],
                            acc_sh.at[pl.ds(sid * half, half)])
            plsc.subcore_barrier()

            pltpu.sync_copy(acc_sh.at[pl.ds(sid * nps2, nps2)], res_v)

            def red_loop(s, _):
                pltpu.sync_copy(
                    acc_sh.at[pl.ds(s * half + sid * nps2, nps2)], tmp_v)

                def add_body(i, _):
                    sl = pl.ds(i * LANES, LANES)
                    res_v[sl] = res_v[sl] + tmp_v[sl]
                    return 0
                lax.fori_loop(0, nps2 // LANES, add_body, 0, unroll=8)
                return 0
            lax.fori_loop(1, NS, red_loop, 0)

            pltpu.sync_copy(
                res_v,
                partial_hbm.at[pl.ds(cid * n_pad + w * half + sid * nps2,
                                     nps2)])
            return 0
        lax.fori_loop(0, 8, wave_body, 0)

    return body, n_pad


def _combine_kernel(p_ref, o_ref):
    o_ref[...] = (p_ref[0] + p_ref[1]) * FACTOR


def kernel(edge_index, atom_type, edge_J, edge_spin_distance):
    N = atom_type.shape[0]
    E = edge_J.shape[0]
    center = edge_index[0]
    j_flat = edge_J.reshape(E)

    sc_fn, n_pad = _sc_scatter_kernel(E, N)
    eng_flat, partial = sc_fn(center, j_flat, edge_spin_distance)

    p3 = partial.reshape(NC, n_pad // 128, 128)
    atom_pad = pl.pallas_call(
        _combine_kernel,
        out_shape=jax.ShapeDtypeStruct((n_pad // 128, 128), jnp.float32),
    )(p3)
    atom_eng = atom_pad.reshape(n_pad)[:N].reshape(N, 1)
    return eng_flat.reshape(E, 1), atom_eng


# speedup vs baseline: 1.8176x; 1.8176x over previous
import functools
import math

import jax
import jax.numpy as jnp
from jax import lax
from jax.experimental import pallas as pl
from jax.experimental.pallas import tpu as pltpu
from jax.experimental.pallas import tpu_sc as plsc

AVG_NUM_NEIGHBORS = 64.0
FACTOR = 1.0 / math.sqrt(AVG_NUM_NEIGHBORS)

NC = 2
NS = 16
NW = NC * NS
LANES = 16
ROW = 128
ROWS = 80
CHUNK = ROW * ROWS


def _sc_scatter_kernel(E, N):
    assert E % ROW == 0
    e_rows = E // ROW
    n_chunks = -(-e_rows // ROWS)
    cpw = -(-n_chunks // NW)
    cpw += cpw % 2
    nps = -(-N // (NS * 8)) * 8
    n_pad = nps * NS

    mesh = plsc.VectorSubcoreMesh(core_axis_name="c", subcore_axis_name="s")

    @functools.partial(
        pl.kernel,
        out_type=(
            jax.ShapeDtypeStruct((E,), jnp.float32),
            jax.ShapeDtypeStruct((NC * n_pad,), jnp.float32),
        ),
        mesh=mesh,
        scratch_types=dict(
            idx0=pltpu.VMEM((ROWS, ROW), jnp.int32),
            idx1=pltpu.VMEM((ROWS, ROW), jnp.int32),
            j0=pltpu.VMEM((CHUNK,), jnp.float32),
            j1=pltpu.VMEM((CHUNK,), jnp.float32),
            s0=pltpu.VMEM((CHUNK,), jnp.float32),
            s1=pltpu.VMEM((CHUNK,), jnp.float32),
            eng0=pltpu.VMEM((CHUNK,), jnp.float32),
            eng1=pltpu.VMEM((CHUNK,), jnp.float32),
            stage_v=pltpu.VMEM((nps,), jnp.float32),
            acc_sh=pltpu.VMEM_SHARED((n_pad,), jnp.float32),
            in_sem=pltpu.SemaphoreType.DMA,
            scat_sem=pltpu.SemaphoreType.DMA,
            wb_sem=pltpu.SemaphoreType.DMA,
        ),
    )
    def body(center_hbm, j_hbm, s_hbm, eng_hbm, partial_hbm,
             idx0, idx1, j0, j1, s0, s1, eng0, eng1, stage_v, acc_sh,
             in_sem, scat_sem, wb_sem):
        cid = lax.axis_index("c")
        sid = lax.axis_index("s")
        wid = sid * NC + cid

        def zero_body(i, _):
            stage_v[pl.ds(i * LANES, LANES)] = jnp.zeros((LANES,), jnp.float32)
            return 0
        lax.fori_loop(0, nps // LANES, zero_body, 0, unroll=8)
        pltpu.sync_copy(stage_v, acc_sh.at[pl.ds(sid * nps, nps)])
        plsc.subcore_barrier()

        def fire_inputs(c, idx_v, j_v, s_v):
            row_off = c * ROWS
            off = c * CHUNK
            pltpu.async_copy(center_hbm.at[pl.ds(row_off, ROWS)], idx_v,
                             in_sem)
            pltpu.async_copy(j_hbm.at[pl.ds(off, CHUNK)], j_v, in_sem)
            pltpu.async_copy(s_hbm.at[pl.ds(off, CHUNK)], s_v, in_sem)

        def wait_inputs(c, idx_v, j_v, s_v):
            row_off = c * ROWS
            off = c * CHUNK
            pltpu.make_async_copy(center_hbm.at[pl.ds(row_off, ROWS)], idx_v,
                                  in_sem).wait()
            pltpu.make_async_copy(j_hbm.at[pl.ds(off, CHUNK)], j_v,
                                  in_sem).wait()
            pltpu.make_async_copy(s_hbm.at[pl.ds(off, CHUNK)], s_v,
                                  in_sem).wait()

        def process(k, idx_v, j_v, s_v, eng_v, nidx, nj, ns_):
            c = wid + k * NW
            cn = wid + (k + 1) * NW

            @pl.when(c < n_chunks)
            def _():
                wait_inputs(c, idx_v, j_v, s_v)

            @pl.when(cn < n_chunks)
            def _():
                fire_inputs(cn, nidx, nj, ns_)

            @pl.when(c < n_chunks)
            def _():
                off = c * CHUNK

                def mul_body(t, _):
                    sl = pl.ds(t * LANES, LANES)
                    eng_v[sl] = j_v[sl] * s_v[sl]
                    return 0
                lax.fori_loop(0, CHUNK // LANES, mul_body, 0, unroll=8)

                pltpu.async_copy(eng_v, eng_hbm.at[pl.ds(off, CHUNK)],
                                 wb_sem)

                def scat_body(r, _):
                    pltpu.async_copy(eng_v.at[pl.ds(r * ROW, ROW)],
                                     acc_sh.at[idx_v.at[r]], scat_sem,
                                     add=True)
                    return 0
                lax.fori_loop(0, ROWS, scat_body, 0)

                def drain_body(r, _):
                    pltpu.make_async_copy(
                        eng_v.at[pl.ds(r * ROW, ROW)],
                        acc_sh.at[idx_v.at[r]], scat_sem).wait()
                    return 0
                lax.fori_loop(0, ROWS, drain_body, 0)
                pltpu.make_async_copy(eng_v, eng_hbm.at[pl.ds(off, CHUNK)],
                                      wb_sem).wait()

        fire_inputs(wid, idx0, j0, s0)

        def pair_body(p, _):
            process(2 * p, idx0, j0, s0, eng0, idx1, j1, s1)
            process(2 * p + 1, idx1, j1, s1, eng1, idx0, j0, s0)
            return 0
        lax.fori_loop(0, cpw // 2, pair_body, 0)

        plsc.subcore_barrier()
        pltpu.sync_copy(acc_sh.at[pl.ds(sid * nps, nps)], stage_v)
        pltpu.sync_copy(stage_v,
                        partial_hbm.at[pl.ds(cid * n_pad + sid * nps, nps)])

    return body, n_pad


def _combine_kernel(p_ref, o_ref):
    o_ref[...] = (p_ref[0] + p_ref[1]) * FACTOR


def kernel(edge_index, atom_type, edge_J, edge_spin_distance):
    N = atom_type.shape[0]
    E = edge_J.shape[0]
    center2d = edge_index[0].reshape(E // ROW, ROW)
    j_flat = edge_J.reshape(E)

    sc_fn, n_pad = _sc_scatter_kernel(E, N)
    eng_flat, partial = sc_fn(center2d, j_flat, edge_spin_distance)

    p3 = partial.reshape(NC, n_pad // 128, 128)
    atom_pad = pl.pallas_call(
        _combine_kernel,
        out_shape=jax.ShapeDtypeStruct((n_pad // 128, 128), jnp.float32),
    )(p3)
    atom_eng = atom_pad.reshape(n_pad)[:N].reshape(N, 1)
    return eng_flat.reshape(E, 1), atom_eng


# speedup vs baseline: 2.8910x; 1.5906x over previous
import functools
import math

import jax
import jax.numpy as jnp
from jax import lax
from jax.experimental import pallas as pl
from jax.experimental.pallas import tpu as pltpu
from jax.experimental.pallas import tpu_sc as plsc

AVG_NUM_NEIGHBORS = 64.0
FACTOR = 1.0 / math.sqrt(AVG_NUM_NEIGHBORS)

NC = 2
NS = 16
NW = NC * NS
LANES = 16
ROW = 128
ROWS = 80
CHUNK = ROW * ROWS


def _sc_scatter_kernel(E, N):
    assert E % ROW == 0
    e_rows = E // ROW
    n_chunks = -(-e_rows // ROWS)
    cpw = -(-n_chunks // NW)
    cpw += cpw % 2
    nps = -(-N // (NS * 8)) * 8
    n_pad = nps * NS

    mesh = plsc.VectorSubcoreMesh(core_axis_name="c", subcore_axis_name="s")

    @functools.partial(
        pl.kernel,
        out_type=(
            jax.ShapeDtypeStruct((E,), jnp.float32),
            jax.ShapeDtypeStruct((NC * n_pad,), jnp.float32),
        ),
        mesh=mesh,
        scratch_types=dict(
            idx0=pltpu.VMEM((ROWS, ROW), jnp.int32),
            idx1=pltpu.VMEM((ROWS, ROW), jnp.int32),
            j0=pltpu.VMEM((CHUNK,), jnp.float32),
            j1=pltpu.VMEM((CHUNK,), jnp.float32),
            s0=pltpu.VMEM((CHUNK,), jnp.float32),
            s1=pltpu.VMEM((CHUNK,), jnp.float32),
            eng0=pltpu.VMEM((CHUNK,), jnp.float32),
            eng1=pltpu.VMEM((CHUNK,), jnp.float32),
            stage_v=pltpu.VMEM((nps,), jnp.float32),
            acc_sh=pltpu.VMEM_SHARED((n_pad,), jnp.float32),
            in_sem=pltpu.SemaphoreType.DMA,
            scat_sem=pltpu.SemaphoreType.DMA,
            wb_sem=pltpu.SemaphoreType.DMA,
        ),
    )
    def body(center_hbm, j_hbm, s_hbm, eng_hbm, partial_hbm,
             idx0, idx1, j0, j1, s0, s1, eng0, eng1, stage_v, acc_sh,
             in_sem, scat_sem, wb_sem):
        cid = lax.axis_index("c")
        sid = lax.axis_index("s")
        wid = sid * NC + cid

        def zero_body(i, _):
            stage_v[pl.ds(i * LANES, LANES)] = jnp.zeros((LANES,), jnp.float32)
            return 0
        lax.fori_loop(0, nps // LANES, zero_body, 0, unroll=8)
        pltpu.sync_copy(stage_v, acc_sh.at[pl.ds(sid * nps, nps)])
        plsc.subcore_barrier()

        def fire_inputs(c, idx_v, j_v, s_v):
            row_off = c * ROWS
            off = c * CHUNK
            pltpu.async_copy(center_hbm.at[pl.ds(row_off, ROWS)], idx_v,
                             in_sem)
            pltpu.async_copy(j_hbm.at[pl.ds(off, CHUNK)], j_v, in_sem)
            pltpu.async_copy(s_hbm.at[pl.ds(off, CHUNK)], s_v, in_sem)

        def wait_inputs(c, idx_v, j_v, s_v):
            row_off = c * ROWS
            off = c * CHUNK
            pltpu.make_async_copy(center_hbm.at[pl.ds(row_off, ROWS)], idx_v,
                                  in_sem).wait()
            pltpu.make_async_copy(j_hbm.at[pl.ds(off, CHUNK)], j_v,
                                  in_sem).wait()
            pltpu.make_async_copy(s_hbm.at[pl.ds(off, CHUNK)], s_v,
                                  in_sem).wait()

        def process(k, idx_v, j_v, s_v, eng_v, nidx, nj, ns_):
            c = wid + k * NW
            cn = wid + (k + 1) * NW

            @pl.when(c < n_chunks)
            def _():
                wait_inputs(c, idx_v, j_v, s_v)

            @pl.when(cn < n_chunks)
            def _():
                fire_inputs(cn, nidx, nj, ns_)

            @pl.when(c < n_chunks)
            def _():
                off = c * CHUNK

                def row_body(r, _):
                    for t in range(ROW // LANES):
                        sl = pl.ds(r * ROW + t * LANES, LANES)
                        eng_v[sl] = j_v[sl] * s_v[sl]
                    pltpu.async_copy(eng_v.at[pl.ds(r * ROW, ROW)],
                                     acc_sh.at[idx_v.at[r]], scat_sem,
                                     add=True)
                    return 0
                lax.fori_loop(0, ROWS, row_body, 0)

                pltpu.async_copy(eng_v, eng_hbm.at[pl.ds(off, CHUNK)],
                                 wb_sem)

                def drain_body(r, _):
                    pltpu.make_async_copy(
                        eng_v.at[pl.ds(r * ROW, ROW)],
                        acc_sh.at[idx_v.at[r]], scat_sem).wait()
                    return 0
                lax.fori_loop(0, ROWS, drain_body, 0)
                pltpu.make_async_copy(eng_v, eng_hbm.at[pl.ds(off, CHUNK)],
                                      wb_sem).wait()

        fire_inputs(wid, idx0, j0, s0)

        def pair_body(p, _):
            process(2 * p, idx0, j0, s0, eng0, idx1, j1, s1)
            process(2 * p + 1, idx1, j1, s1, eng1, idx0, j0, s0)
            return 0
        lax.fori_loop(0, cpw // 2, pair_body, 0)

        plsc.subcore_barrier()
        pltpu.sync_copy(acc_sh.at[pl.ds(sid * nps, nps)], stage_v)
        pltpu.sync_copy(stage_v,
                        partial_hbm.at[pl.ds(cid * n_pad + sid * nps, nps)])

    return body, n_pad


def _combine_kernel(p_ref, o_ref):
    o_ref[...] = (p_ref[0] + p_ref[1]) * FACTOR


def kernel(edge_index, atom_type, edge_J, edge_spin_distance):
    N = atom_type.shape[0]
    E = edge_J.shape[0]
    center2d = edge_index[0].reshape(E // ROW, ROW)
    j_flat = edge_J.reshape(E)

    sc_fn, n_pad = _sc_scatter_kernel(E, N)
    eng_flat, partial = sc_fn(center2d, j_flat, edge_spin_distance)

    p3 = partial.reshape(NC, n_pad // 128, 128)
    atom_pad = pl.pallas_call(
        _combine_kernel,
        out_shape=jax.ShapeDtypeStruct((n_pad // 128, 128), jnp.float32),
    )(p3)
    atom_eng = atom_pad.reshape(n_pad)[:N].reshape(N, 1)
    return eng_flat.reshape(E, 1), atom_eng
